# baseline (speedup 1.0000x reference)
import functools

import jax
import jax.numpy as jnp
from jax import lax
from jax.experimental import pallas as pl
from jax.experimental.pallas import tpu as pltpu
from jax.experimental.pallas import tpu_sc as plsc

N = 2_000_000
BATCH = 16
RES = 128
NCELLS = BATCH * RES * RES * RES
G = NCELLS // (32 * 128)
NC = 2
NS = 16
NW = NC * NS
C = 16000
H = C // 2
NCHUNK = N // C
LANES = 16


def _sc_idx_body(ptst_hbm, bidx_hbm, pk_hbm,
                 px_v, py_v, pz_v, pk_v, sem_in):
    wid = lax.axis_index("s") * NC + lax.axis_index("c")
    n_my = (NCHUNK - wid + NW - 1) // NW

    def chunk_body(i, carry):
        base = (wid + i * NW) * C
        cp1 = pltpu.async_copy(ptst_hbm.at[pl.ds(0, 1), pl.ds(base, C)],
                               px_v, sem_in)
        cp2 = pltpu.async_copy(ptst_hbm.at[pl.ds(1, 1), pl.ds(base, C)],
                               py_v, sem_in)
        cp3 = pltpu.async_copy(ptst_hbm.at[pl.ds(2, 1), pl.ds(base, C)],
                               pz_v, sem_in)
        cp4 = pltpu.async_copy(bidx_hbm.at[pl.ds(base, C)], pk_v, sem_in)
        cp1.wait()
        cp2.wait()
        cp3.wait()
        cp4.wait()

        def idx_body(j, carry2):
            sl = pl.ds(j * LANES, LANES)
            x = px_v[0, sl]
            y = py_v[0, sl]
            z = pz_v[0, sl]
            b = pk_v[sl]
            scale = jnp.float32(RES)
            gx = ((x * 0.5 + 0.5) * scale).astype(jnp.int32)
            gy = ((y * 0.5 + 0.5) * scale).astype(jnp.int32)
            gz = ((z * 0.5 + 0.5) * scale).astype(jnp.int32)
            gx = jnp.minimum(jnp.maximum(gx, 0), RES - 1)
            gy = jnp.minimum(jnp.maximum(gy, 0), RES - 1)
            gz = jnp.minimum(jnp.maximum(gz, 0), RES - 1)
            widx = (lax.shift_left(b * 512 + gx * 4
                                   + lax.shift_right_logical(gy, 5), 7) | gz)
            pk_v[sl] = lax.shift_left(widx, 5) | (gy & 31)
            return carry2

        lax.fori_loop(0, C // LANES, idx_body, 0)
        pltpu.sync_copy(pk_v, pk_hbm.at[pl.ds(base, C)])
        return carry

    lax.fori_loop(0, n_my, chunk_body, 0)


def _sc_gather_body(pk_hbm, grid_hbm, out_hbm,
                    pk0_v, pk1_v, widx_v, gath_v, grid_sh,
                    sem_pk0, sem_pk1, sem_g1, sem_g2, sem_g3, sem_g4,
                    sem_out, sem_grid):
    wid = lax.axis_index("s") * NC + lax.axis_index("c")
    sid = lax.axis_index("s")
    n_my = (NCHUNK - wid + NW - 1) // NW
    pk_bufs = (pk0_v, pk1_v)
    pk_sems = (sem_pk0, sem_pk1)

    @pl.when(sid == 0)
    def _():
        pltpu.sync_copy(grid_hbm, grid_sh)

    pltpu.async_copy(pk_hbm.at[pl.ds(wid * C, C)], pk0_v, sem_pk0).wait()
    plsc.subcore_barrier()

    UNROLL = 8

    def unpack_half(pk_v, lo, hi):
        def unpack_body(j, carry2):
            for u in range(UNROLL):
                sl = pl.ds(j * LANES * UNROLL + u * LANES, LANES)
                widx_v[sl] = lax.shift_right_logical(pk_v[sl], 5)
            return carry2

        lax.fori_loop(lo // (LANES * UNROLL), hi // (LANES * UNROLL),
                      unpack_body, 0)

    def out_half(pk_v, lo, hi):
        def out_body(j, carry2):
            for u in range(UNROLL):
                sl = pl.ds(j * LANES * UNROLL + u * LANES, LANES)
                gath_v[sl] = lax.shift_right_logical(
                    gath_v[sl], pk_v[sl] & 31) & 1
            return carry2

        lax.fori_loop(lo // (LANES * UNROLL), hi // (LANES * UNROLL),
                      out_body, 0)

    def pair_body(ii, carry):
        for b in range(2):
            i = ii * 2 + b

            @pl.when(i < n_my)
            def _():
                base = (wid + i * NW) * C
                pk_v = pk_bufs[b]
                nxt = (wid + (i + 1) * NW) * C

                @pl.when(i + 1 < n_my)
                def _():
                    pltpu.make_async_copy(
                        pk_hbm.at[pl.ds(nxt, C)], pk_bufs[1 - b],
                        pk_sems[1 - b]).start()

                Q = C // 4
                unpack_half(pk_v, 0, Q)
                @pl.when(i > 0)
                def _():
                    pltpu.make_async_copy(
                        gath_v, out_hbm.at[pl.ds(base, C)], sem_out).wait()

                g1 = pltpu.async_copy(grid_sh.at[widx_v.at[pl.ds(0, Q)]],
                                      gath_v.at[pl.ds(0, Q)], sem_g1)
                unpack_half(pk_v, Q, 2 * Q)
                g2 = pltpu.async_copy(grid_sh.at[widx_v.at[pl.ds(Q, Q)]],
                                      gath_v.at[pl.ds(Q, Q)], sem_g2)
                unpack_half(pk_v, 2 * Q, 3 * Q)
                g3 = pltpu.async_copy(
                    grid_sh.at[widx_v.at[pl.ds(2 * Q, Q)]],
                    gath_v.at[pl.ds(2 * Q, Q)], sem_g3)
                unpack_half(pk_v, 3 * Q, C)
                g4 = pltpu.async_copy(
                    grid_sh.at[widx_v.at[pl.ds(3 * Q, Q)]],
                    gath_v.at[pl.ds(3 * Q, Q)], sem_g4)
                g1.wait()
                out_half(pk_v, 0, Q)
                g2.wait()
                out_half(pk_v, Q, 2 * Q)
                g3.wait()
                out_half(pk_v, 2 * Q, 3 * Q)
                g4.wait()
                out_half(pk_v, 3 * Q, C)
                pltpu.make_async_copy(
                    gath_v, out_hbm.at[pl.ds(base, C)], sem_out).start()

                @pl.when(i + 1 < n_my)
                def _():
                    pltpu.make_async_copy(
                        pk_hbm.at[pl.ds(nxt, C)], pk_bufs[1 - b],
                        pk_sems[1 - b]).wait()
        return carry

    lax.fori_loop(0, (NCHUNK + NW - 1) // NW // 2 + 1, pair_body, 0)

    @pl.when(n_my > 0)
    def _():
        last_base = (wid + (n_my - 1) * NW) * C
        pltpu.make_async_copy(
            gath_v, out_hbm.at[pl.ds(last_base, C)], sem_out).wait()


@jax.jit
def _run(ptst, bidx32, grid_words):
    mesh = plsc.VectorSubcoreMesh(core_axis_name="c", subcore_axis_name="s")
    idx_k = functools.partial(
        pl.kernel,
        out_type=jax.ShapeDtypeStruct((N,), jnp.int32),
        mesh=mesh,
        scratch_types=[
            pltpu.VMEM((1, C), jnp.float32),
            pltpu.VMEM((1, C), jnp.float32),
            pltpu.VMEM((1, C), jnp.float32),
            pltpu.VMEM((C,), jnp.int32),
            pltpu.SemaphoreType.DMA,
        ],
    )(_sc_idx_body)
    pk = idx_k(ptst, bidx32)
    gather_k = functools.partial(
        pl.kernel,
        out_type=jax.ShapeDtypeStruct((N,), jnp.int32),
        mesh=mesh,
        scratch_types=[
            pltpu.VMEM((C,), jnp.int32),
            pltpu.VMEM((C,), jnp.int32),
            pltpu.VMEM((C,), jnp.int32),
            pltpu.VMEM((C,), jnp.int32),
            pltpu.VMEM_SHARED((NCELLS // 32,), jnp.int32),
            pltpu.SemaphoreType.DMA,
            pltpu.SemaphoreType.DMA,
            pltpu.SemaphoreType.DMA,
            pltpu.SemaphoreType.DMA,
            pltpu.SemaphoreType.DMA,
            pltpu.SemaphoreType.DMA,
            pltpu.SemaphoreType.DMA,
            pltpu.SemaphoreType.DMA,
        ],
    )(_sc_gather_body)
    return gather_k(pk, grid_words)


def kernel(pts, bidx, occ_grid_per_batch):
    occ3d = occ_grid_per_batch.reshape(G, 32, 128)
    weights = lax.shift_left(
        jnp.int32(1), lax.broadcasted_iota(jnp.int32, (1, 32, 1), 1))
    grid_words = jnp.sum(
        jnp.where(occ3d[:, :, :], weights, 0), axis=1,
        dtype=jnp.int32).reshape(-1)
    out = _run(pts.T, bidx.astype(jnp.int32), grid_words)
    return out.astype(bool)
